# baseline (speedup 1.0000x reference)
import functools

import jax
import jax.numpy as jnp
from jax import lax
from jax.experimental import pallas as pl
from jax.experimental.pallas import tpu as pltpu
from jax.experimental.pallas import tpu_sc as plsc

NUM_USERS = 1000000
NUM_ITEMS = 1000000
EMBED_DIM = 32
BATCH = 16384

_INFO = plsc.get_sparse_core_info()
_NC, _NS, _L = _INFO.num_cores, _INFO.num_subcores, _INFO.num_lanes
_NW = _NC * _NS
_BPW = BATCH // _NW
_GROUPS = _BPW // _L

_mesh = plsc.VectorSubcoreMesh(core_axis_name="c", subcore_axis_name="s")


@functools.partial(
    pl.kernel,
    mesh=_mesh,
    out_type=jax.ShapeDtypeStruct((BATCH,), jnp.float32),
    compiler_params=pltpu.CompilerParams(needs_layout_passes=False,
                                         use_tc_tiling_on_sc=False),
    scratch_types=[
        pltpu.VMEM((_BPW,), jnp.int32),
        pltpu.VMEM((_BPW,), jnp.int32),
        pltpu.VMEM((_BPW, EMBED_DIM), jnp.float32),
        pltpu.VMEM((_BPW, EMBED_DIM), jnp.float32),
        pltpu.VMEM((_BPW,), jnp.float32),
        pltpu.VMEM((_BPW,), jnp.float32),
        pltpu.VMEM((_L,), jnp.float32),
        pltpu.VMEM((_BPW,), jnp.float32),
        pltpu.SemaphoreType.DMA,
    ],
)
def _mf_sc(uid_hbm, iid_hbm, uemb_hbm, iemb_hbm, ub_hbm, ib_hbm, gb_hbm,
           out_hbm, uidx_v, iidx_v, urow_v, irow_v, ub_v, ib_v, gb_v,
           out_v, sem):
    wid = lax.axis_index("s") * _NC + lax.axis_index("c")
    base = wid * _BPW

    pltpu.sync_copy(uid_hbm.at[pl.ds(base, _BPW)], uidx_v)
    pltpu.sync_copy(iid_hbm.at[pl.ds(base, _BPW)], iidx_v)
    pltpu.sync_copy(gb_hbm, gb_v)

    d1 = pltpu.async_copy(uemb_hbm.at[uidx_v], urow_v, sem)
    d2 = pltpu.async_copy(iemb_hbm.at[iidx_v], irow_v, sem)
    d3 = pltpu.async_copy(ub_hbm.at[uidx_v], ub_v, sem)
    d4 = pltpu.async_copy(ib_hbm.at[iidx_v], ib_v, sem)
    d1.wait()
    d2.wait()
    d3.wait()
    d4.wait()

    lane = lax.iota(jnp.int32, _L)
    zeros = jnp.zeros((_L,), jnp.int32)
    gb = gb_v[...]

    def group(g, carry):
        b_idx = g * _L + lane
        acc = ub_v[pl.ds(g * _L, _L)] + ib_v[pl.ds(g * _L, _L)] + gb
        for d in range(EMBED_DIM):
            dd = jnp.full((_L,), d, jnp.int32)
            u = plsc.load_gather(urow_v, [b_idx, dd])
            v = plsc.load_gather(irow_v, [b_idx, dd])
            acc = acc + u * v
        out_v[pl.ds(g * _L, _L)] = acc
        return carry

    lax.fori_loop(0, _GROUPS, group, 0)

    pltpu.sync_copy(out_v, out_hbm.at[pl.ds(base, _BPW)])


def kernel(user_ids, item_ids, user_emb, item_emb, user_bias, item_bias,
           global_bias):
    gb16 = jnp.broadcast_to(global_bias.astype(jnp.float32), (_L,))
    return _mf_sc(user_ids.astype(jnp.int32), item_ids.astype(jnp.int32),
                  user_emb, item_emb, user_bias.reshape(-1),
                  item_bias.reshape(-1), gb16)
